# baseline (speedup 1.0000x reference)
import functools

import jax
import jax.numpy as jnp
from jax import lax
from jax.experimental import pallas as pl
from jax.experimental.pallas import tpu as pltpu
from jax.experimental.pallas import tpu_sc as plsc

_VOCAB = 1000000
_DIM = 64
_B = 1024
_S = 200
_NC, _NS = 2, 16
_NW = _NC * _NS
_RPW = _B // _NW
_LANES = 16
_KCH = _DIM // _LANES
_CH0, _CH1 = 104, 96
_NBUF = 4

_mesh = plsc.VectorSubcoreMesh(core_axis_name="c", subcore_axis_name="s")


def _emb_body(x_hbm, table_hbm, pe_hbm, out_hbm,
              idx_v, pe_v, b0, b1, b2, b3,
              gs0, gs1, gs2, gs3, ss0, ss1, ss2, ss3):
    bufs = (b0, b1, b2, b3)
    gsems = (gs0, gs1, gs2, gs3)
    ssems = (ss0, ss1, ss2, ss3)
    wid = lax.axis_index("s") * _NC + lax.axis_index("c")
    ibase = wid * (_RPW * _S)

    pltpu.sync_copy(x_hbm.at[pl.ds(ibase, _RPW * _S)], idx_v)
    pltpu.sync_copy(pe_hbm, pe_v)

    def issue_gather(r, j):
        i0 = r * _S
        pltpu.async_copy(table_hbm.at[idx_v.at[pl.ds(i0, _CH0)]],
                         bufs[j].at[pl.ds(0, _CH0)], gsems[j])
        pltpu.async_copy(table_hbm.at[idx_v.at[pl.ds(i0 + _CH0, _CH1)]],
                         bufs[j].at[pl.ds(_CH0, _CH1)], gsems[j])

    def wait_gather(r, j):
        i0 = r * _S
        pltpu.make_async_copy(table_hbm.at[idx_v.at[pl.ds(i0, _CH0)]],
                              bufs[j].at[pl.ds(0, _CH0)], gsems[j]).wait()
        pltpu.make_async_copy(table_hbm.at[idx_v.at[pl.ds(i0 + _CH0, _CH1)]],
                              bufs[j].at[pl.ds(_CH0, _CH1)], gsems[j]).wait()

    def issue_scatter(r, j):
        pltpu.async_copy(bufs[j], out_hbm.at[pl.ds(ibase + r * _S, _S)],
                         ssems[j])

    def wait_scatter(r, j):
        pltpu.make_async_copy(bufs[j], out_hbm.at[pl.ds(ibase + r * _S, _S)],
                              ssems[j]).wait()

    def add_pe(j):
        buf = bufs[j]

        def body(rr, carry):
            for k in range(_KCH):
                sl = pl.ds(k * _LANES, _LANES)
                plsc.addupdate(buf.at[rr, sl], pe_v[rr, sl])
            return carry

        lax.fori_loop(0, _S, body, 0, unroll=4)

    issue_gather(0, 0)
    issue_gather(1, 1)

    def loop_body(g, carry):
        for j in range(_NBUF):
            r = 4 * g + j
            jn = (j + 2) % _NBUF
            @pl.when(r >= 2)
            def _():
                wait_scatter(r - 2, jn)

            @pl.when(r + 2 < _RPW)
            def _():
                issue_gather(r + 2, jn)

            wait_gather(r, j)
            add_pe(j)
            issue_scatter(r, j)
        return carry

    lax.fori_loop(0, _RPW // _NBUF, loop_body, 0)

    for r in range(_RPW - 2, _RPW):
        wait_scatter(r, r % _NBUF)


def _pos_encoding(seq_len, dim):
    pos = jnp.arange(seq_len, dtype=jnp.float32)[:, None]
    i = jnp.arange(0, dim, 2, dtype=jnp.float32)
    div = jnp.exp(-jnp.log(10000.0) * i / dim)
    ang = pos * div[None, :]
    pe = jnp.zeros((seq_len, dim), dtype=jnp.float32)
    pe = pe.at[:, 0::2].set(jnp.sin(ang))
    pe = pe.at[:, 1::2].set(jnp.cos(ang))
    return pe


@functools.partial(
    pl.kernel,
    out_type=jax.ShapeDtypeStruct((_B * _S, _DIM), jnp.float32),
    mesh=_mesh,
    compiler_params=pltpu.CompilerParams(use_tc_tiling_on_sc=False),
    scratch_types=[
        pltpu.VMEM((_RPW * _S,), jnp.int32),
        pltpu.VMEM((_S, _DIM), jnp.float32),
        pltpu.VMEM((_S, _DIM), jnp.float32),
        pltpu.VMEM((_S, _DIM), jnp.float32),
        pltpu.VMEM((_S, _DIM), jnp.float32),
        pltpu.VMEM((_S, _DIM), jnp.float32),
        pltpu.SemaphoreType.DMA,
        pltpu.SemaphoreType.DMA,
        pltpu.SemaphoreType.DMA,
        pltpu.SemaphoreType.DMA,
        pltpu.SemaphoreType.DMA,
        pltpu.SemaphoreType.DMA,
        pltpu.SemaphoreType.DMA,
        pltpu.SemaphoreType.DMA,
    ],
)
def _emb_kernel(x_hbm, table_hbm, pe_hbm, out_hbm, *rest):
    _emb_body(x_hbm, table_hbm, pe_hbm, out_hbm, *rest)


def kernel(x, token_table):
    pe = _pos_encoding(_S, _DIM)
    xf = x.reshape(-1).astype(jnp.int32)
    out = _emb_kernel(xf, token_table, pe)
    return out.reshape(_B, _S, _DIM)


# speedup vs baseline: 1.0561x; 1.0561x over previous
import functools

import jax
import jax.numpy as jnp
from jax import lax
from jax.experimental import pallas as pl
from jax.experimental.pallas import tpu as pltpu
from jax.experimental.pallas import tpu_sc as plsc

_VOCAB = 1000000
_DIM = 64
_WIDE = 128
_B = 1024
_S = 200
_NC, _NS = 2, 16
_NW = _NC * _NS
_RPW = _B // _NW
_LANES = 16
_KCH = _DIM // _LANES
_CH = (104, 96)
_NBUF = 4
_HALVES = 2 * _RPW

_mesh = plsc.VectorSubcoreMesh(core_axis_name="c", subcore_axis_name="s")


def _emb_body(x_hbm, table_hbm, pe_hbm, out_hbm,
              idx_v, pe_v, b0, b1, b2, b3, o0, o1,
              gs0, gs1, gs2, gs3, ss0, ss1):
    bufs = (b0, b1, b2, b3)
    obufs = (o0, o1)
    gsems = (gs0, gs1, gs2, gs3)
    ssems = (ss0, ss1)
    wid = lax.axis_index("s") * _NC + lax.axis_index("c")
    ibase = wid * (_RPW * _S)

    pltpu.sync_copy(x_hbm.at[pl.ds(ibase, _RPW * _S)], idx_v)
    pltpu.sync_copy(pe_hbm, pe_v)


    def issue_gather(h, j, off, L):
        i0 = (h // 2) * _S + off
        pltpu.async_copy(table_hbm.at[idx_v.at[pl.ds(i0, L)]],
                         bufs[j].at[pl.ds(0, L)], gsems[j])

    def wait_gather(h, j, off, L):
        i0 = (h // 2) * _S + off
        pltpu.make_async_copy(table_hbm.at[idx_v.at[pl.ds(i0, L)]],
                              bufs[j].at[pl.ds(0, L)], gsems[j]).wait()

    def issue_scatter(h, o, off, L):
        r0 = ibase + (h // 2) * _S + off
        pltpu.async_copy(obufs[o].at[pl.ds(0, L)],
                         out_hbm.at[pl.ds(r0, L)], ssems[o])

    def wait_scatter(h, o, off, L):
        r0 = ibase + (h // 2) * _S + off
        pltpu.make_async_copy(obufs[o].at[pl.ds(0, L)],
                              out_hbm.at[pl.ds(r0, L)], ssems[o]).wait()

    def add_pe(j, o, off, L):
        buf = bufs[j]
        obuf = obufs[o]

        def body(rr, carry):
            for k in range(_KCH):
                sl = pl.ds(k * _LANES, _LANES)
                obuf[rr, sl] = buf[rr, sl] + pe_v[off + rr, sl]
            return carry

        lax.fori_loop(0, L, body, 0, unroll=4)

    def geom(j):
        off = 0 if j % 2 == 0 else _CH[0]
        return off, _CH[j % 2]

    issue_gather(0, 0, *geom(0))
    issue_gather(1, 1, *geom(1))

    def loop_body(g, carry):
        for j in range(_NBUF):
            h = _NBUF * g + j
            off, L = geom(j)
            o = j % 2
            jn = (j + 2) % _NBUF
            @pl.when(h + 2 < _HALVES)
            def _():
                issue_gather(h + 2, jn, *geom(jn))

            wait_gather(h, j, off, L)
            @pl.when(h >= 2)
            def _():
                wait_scatter(h - 2, o, *geom(o))
            add_pe(j, o, off, L)
            issue_scatter(h, o, off, L)
        return carry

    lax.fori_loop(0, _HALVES // _NBUF, loop_body, 0)

    for h in range(_HALVES - 2, _HALVES):
        wait_scatter(h, h % 2, *geom(h % 2))


def _pos_encoding(seq_len, dim):
    pos = jnp.arange(seq_len, dtype=jnp.float32)[:, None]
    i = jnp.arange(0, dim, 2, dtype=jnp.float32)
    div = jnp.exp(-jnp.log(10000.0) * i / dim)
    ang = pos * div[None, :]
    pe = jnp.zeros((seq_len, dim), dtype=jnp.float32)
    pe = pe.at[:, 0::2].set(jnp.sin(ang))
    pe = pe.at[:, 1::2].set(jnp.cos(ang))
    return pe


@functools.partial(
    pl.kernel,
    out_type=jax.ShapeDtypeStruct((_B * _S, _DIM), jnp.float32),
    mesh=_mesh,
    compiler_params=pltpu.CompilerParams(use_tc_tiling_on_sc=True),
    scratch_types=[
        pltpu.VMEM((_RPW * _S,), jnp.int32),
        pltpu.VMEM((_S, _DIM), jnp.float32),
        pltpu.VMEM((_CH[0], _WIDE), jnp.float32),
        pltpu.VMEM((_CH[0], _WIDE), jnp.float32),
        pltpu.VMEM((_CH[0], _WIDE), jnp.float32),
        pltpu.VMEM((_CH[0], _WIDE), jnp.float32),
        pltpu.VMEM((_CH[0], _DIM), jnp.float32),
        pltpu.VMEM((_CH[0], _DIM), jnp.float32),
        pltpu.SemaphoreType.DMA,
        pltpu.SemaphoreType.DMA,
        pltpu.SemaphoreType.DMA,
        pltpu.SemaphoreType.DMA,
        pltpu.SemaphoreType.DMA,
        pltpu.SemaphoreType.DMA,
    ],
)
def _emb_kernel(x_hbm, table_hbm, pe_hbm, out_hbm, *rest):
    _emb_body(x_hbm, table_hbm, pe_hbm, out_hbm, *rest)


def kernel(x, token_table):
    pe = _pos_encoding(_S, _DIM)
    xf = x.reshape(-1).astype(jnp.int32)
    tw = jnp.pad(token_table, ((0, 0), (0, _WIDE - _DIM)))
    out = _emb_kernel(xf, tw, pe)
    return out.reshape(_B, _S, _DIM)
